# baseline (speedup 1.0000x reference)
import functools

import jax
import jax.numpy as jnp
import numpy as np
from jax import lax
from jax.experimental import pallas as pl
from jax.experimental.pallas import tpu as pltpu
from jax.experimental.pallas import tpu_sc as plsc

_X_TILES = 4
_Y_TILES = 4
_NWORKERS = 32
_ROWS = 16
_LANES = 16


@functools.lru_cache(maxsize=None)
def _inv_perm_table(B):
    with jax.ensure_compile_time_eval():
        u = jax.random.uniform(jax.random.key(42), (B, _Y_TILES * _X_TILES))
        perm = np.asarray(jnp.argsort(u, axis=-1))
    return np.argsort(perm, axis=-1).astype(np.int32)


def kernel(image):
    B, C, H, W = image.shape
    hs, ws = _Y_TILES, _X_TILES
    h, w = H // hs, W // ws
    ntiles = hs * ws
    nq = h // _ROWS
    nitems = B * C * nq
    items_per_worker = nitems // _NWORKERS
    ngroups = w // _LANES

    inv = _inv_perm_table(B).reshape(-1)
    inv = np.concatenate([inv, np.zeros(_LANES, np.int32)])
    inv_tab = jnp.asarray(inv)

    mesh = plsc.VectorSubcoreMesh(core_axis_name="core", subcore_axis_name="subcore")

    @functools.partial(
        pl.kernel,
        out_type=jax.ShapeDtypeStruct((B, C, H, W), image.dtype),
        mesh=mesh,
        scratch_types=[
            pltpu.VMEM((B * ntiles + _LANES,), jnp.int32),
            pltpu.VMEM((hs, _ROWS, W), jnp.float32),
            pltpu.VMEM((hs, _ROWS, W), jnp.float32),
            pltpu.VMEM((hs, _ROWS, W), jnp.float32),
            pltpu.VMEM((hs, _ROWS, W), jnp.float32),
            pltpu.SemaphoreType.DMA,
            pltpu.SemaphoreType.DMA,
            pltpu.SemaphoreType.DMA,
            pltpu.SemaphoreType.DMA,
            pltpu.SemaphoreType.DMA,
            pltpu.SemaphoreType.DMA,
        ],
    )
    def assemble(x_hbm, t_hbm, o_hbm, t_vmem, in_a, in_b, out_a, out_b,
                 isem, gs_a, gs_b, ss_a, ss_b, lsem):
        wid = lax.axis_index("subcore") * 2 + lax.axis_index("core")
        ins = (in_a, in_b)
        outs = (out_a, out_b)
        gsems = (gs_a, gs_b)
        ssems = (ss_a, ss_b)

        pltpu.async_copy(t_hbm, t_vmem, isem).wait()
        base = wid * items_per_worker

        def item_coords(k):
            it = base + k
            b = it // (C * nq)
            r = it % (C * nq)
            return b, r // nq, r % nq

        def load_copies(k, s):
            b, c, q = item_coords(k)
            return [
                pltpu.make_async_copy(
                    x_hbm.at[b, c, pl.ds(si * h + q * _ROWS, _ROWS), :],
                    ins[s].at[si],
                    gsems[s],
                )
                for si in range(hs)
            ]

        def store_copies(k, s):
            b, c, q = item_coords(k)
            return [
                pltpu.make_async_copy(
                    outs[s].at[ti],
                    o_hbm.at[b, c, pl.ds(ti * h + q * _ROWS, _ROWS), :],
                    ssems[s],
                )
                for ti in range(hs)
            ]

        def assemble_item(k, s):
            b, _, _ = item_coords(k)

            @pl.loop(0, ntiles)
            def _(u):
                t = t_vmem[pl.ds(b * ntiles + u, _LANES)][0]
                ti = t // ws
                dst_col = (t % ws) * w
                si = u // ws
                src_col = (u % ws) * w

                @pl.loop(0, _ROWS, step=8)
                def _(y0):
                    for dy in range(8):
                        for m in range(ngroups):
                            outs[s][ti, y0 + dy, pl.ds(dst_col + m * _LANES, _LANES)] = (
                                ins[s][si, y0 + dy, pl.ds(src_col + m * _LANES, _LANES)]
                            )

        for cp in load_copies(0, 0):
            cp.start()

        @pl.loop(0, items_per_worker, step=2)
        def _(k0):
            for s in range(2):
                k = k0 + s
                for cp in load_copies(k, s):
                    cp.wait()

                @pl.when(k >= 1)
                def _():
                    for cp in store_copies(k - 1, 1 - s):
                        cp.wait()

                @pl.when(k + 1 < items_per_worker)
                def _():
                    for cp in load_copies(k + 1, 1 - s):
                        cp.start()

                assemble_item(k, s)
                for cp in store_copies(k, s):
                    cp.start()

        last = items_per_worker - 1
        for cp in store_copies(last, last % 2):
            cp.wait()

    return assemble(image, inv_tab)


# speedup vs baseline: 1.0295x; 1.0295x over previous
import functools

import jax
import jax.numpy as jnp
import numpy as np
from jax import lax
from jax.experimental import pallas as pl
from jax.experimental.pallas import tpu as pltpu
from jax.experimental.pallas import tpu_sc as plsc

_X_TILES = 4
_Y_TILES = 4
_NWORKERS = 32
_ROWS = 16
_LANES = 16


@functools.lru_cache(maxsize=None)
def _inv_perm_table(B):
    with jax.ensure_compile_time_eval():
        u = jax.random.uniform(jax.random.key(42), (B, _Y_TILES * _X_TILES))
        perm = np.asarray(jnp.argsort(u, axis=-1))
    return np.argsort(perm, axis=-1).astype(np.int32)


def kernel(image):
    B, C, H, W = image.shape
    hs, ws = _Y_TILES, _X_TILES
    h, w = H // hs, W // ws
    ntiles = hs * ws
    nq = h // _ROWS
    nitems = B * C * nq
    items_per_worker = nitems // _NWORKERS
    ngroups = w // _LANES

    inv = _inv_perm_table(B).reshape(-1)
    inv = np.concatenate([inv, np.zeros(_LANES, np.int32)])
    inv_tab = jnp.asarray(inv)

    mesh = plsc.VectorSubcoreMesh(core_axis_name="core", subcore_axis_name="subcore")

    @functools.partial(
        pl.kernel,
        out_type=jax.ShapeDtypeStruct((B, C, H, W), image.dtype),
        mesh=mesh,
        scratch_types=[
            pltpu.VMEM((B * ntiles + _LANES,), jnp.int32),
            pltpu.VMEM((hs, _ROWS, W), jnp.float32),
            pltpu.VMEM((hs, _ROWS, W), jnp.float32),
            pltpu.VMEM((hs, _ROWS, W), jnp.float32),
            pltpu.VMEM((hs, _ROWS, W), jnp.float32),
            pltpu.SemaphoreType.DMA,
            pltpu.SemaphoreType.DMA,
            pltpu.SemaphoreType.DMA,
            pltpu.SemaphoreType.DMA,
            pltpu.SemaphoreType.DMA,
            pltpu.SemaphoreType.DMA,
        ],
    )
    def assemble(x_hbm, t_hbm, o_hbm, t_vmem, in_a, in_b, out_a, out_b,
                 isem, gs_a, gs_b, ss_a, ss_b, lsem):
        wid = lax.axis_index("subcore") * 2 + lax.axis_index("core")
        ins = (in_a, in_b)
        outs = (out_a, out_b)
        gsems = (gs_a, gs_b)
        ssems = (ss_a, ss_b)

        pltpu.async_copy(t_hbm, t_vmem, isem).wait()
        base = wid * items_per_worker

        def item_coords(k):
            it = base + k
            b = it // (C * nq)
            r = it % (C * nq)
            return b, r // nq, r % nq

        def load_copies(k, s):
            b, c, q = item_coords(k)
            return [
                pltpu.make_async_copy(
                    x_hbm.at[b, c, pl.ds(si * h + q * _ROWS, _ROWS), :],
                    ins[s].at[si],
                    gsems[s],
                )
                for si in range(hs)
            ]

        def store_copies(k, s):
            b, c, q = item_coords(k)
            return [
                pltpu.make_async_copy(
                    outs[s].at[ti],
                    o_hbm.at[b, c, pl.ds(ti * h + q * _ROWS, _ROWS), :],
                    ssems[s],
                )
                for ti in range(hs)
            ]

        def assemble_item(k, s):
            b, _, _ = item_coords(k)

            @pl.loop(0, ntiles)
            def _(u):
                t = t_vmem[pl.ds(b * ntiles + u, _LANES)][0]
                ti = t // ws
                dst_col = (t % ws) * w
                si = u // ws
                src_col = (u % ws) * w

                @pl.loop(0, _ROWS, step=4)
                def _(y0):
                    for dy in range(4):
                        for m in range(ngroups):
                            outs[s][ti, y0 + dy, pl.ds(dst_col + m * _LANES, _LANES)] = (
                                ins[s][si, y0 + dy, pl.ds(src_col + m * _LANES, _LANES)]
                            )

        for cp in load_copies(0, 0):
            cp.start()

        @pl.loop(0, items_per_worker, step=2)
        def _(k0):
            for s in range(2):
                k = k0 + s
                for cp in load_copies(k, s):
                    cp.wait()

                @pl.when(k >= 1)
                def _():
                    for cp in store_copies(k - 1, 1 - s):
                        cp.wait()

                @pl.when(k + 1 < items_per_worker)
                def _():
                    for cp in load_copies(k + 1, 1 - s):
                        cp.start()

                assemble_item(k, s)
                for cp in store_copies(k, s):
                    cp.start()

        last = items_per_worker - 1
        for cp in store_copies(last, last % 2):
            cp.wait()

    return assemble(image, inv_tab)
